# Initial kernel scaffold; baseline (speedup 1.0000x reference)
import jax
import jax.numpy as jnp
from jax.experimental import pallas as pl


def kernel(x, edge_index):
    raise NotImplementedError("write your pallas kernel here")


# speedup vs baseline: 5.3423x; 5.3423x over previous
import jax
import jax.numpy as jnp
from jax import lax
from jax.experimental import pallas as pl
from jax.experimental.pallas import tpu as pltpu
from jax.experimental.pallas import tpu_sc as plsc

N = 10000
D = 128
E = 320000

NUM_CORES = 2
NUM_SUBCORES = 16
LANES = 16
DH = D // NUM_CORES
N_PAD = 10240
ROWS_PER_TILE = N_PAD // NUM_SUBCORES
NB = 64
NBLOCKS = ROWS_PER_TILE // NB
CHUNK = 128
E_PAD = 327680
EDGES_PER_TILE = E_PAD // NUM_SUBCORES
NUM_CHUNKS = EDGES_PER_TILE // CHUNK
CNT_W = 16


def _body(x2, src_hbm, dst_hbm, out_hbm,
          table_sh, acc_sh, cnt_sh,
          src_v, dst_v, rows_v, ones_v, cntblk_v, norm_v, sem):
  cid = lax.axis_index("c")
  sid = lax.axis_index("s")
  base_n = sid * ROWS_PER_TILE
  base_e = sid * EDGES_PER_TILE

  zeros16 = jnp.zeros((LANES,), jnp.float32)
  ones16 = jnp.full((LANES,), 1.0, jnp.float32)

  def zero_norm():
    def zr(r, c):
      for j in range(DH // LANES):
        norm_v[r, pl.ds(j * LANES, LANES)] = zeros16
      return c
    lax.fori_loop(0, NB, zr, 0)

  def zero_acc_slice():
    zero_norm()

    def zb(b, c):
      pltpu.sync_copy(norm_v, acc_sh.at[pl.ds(base_n + b * NB, NB)])
      return c
    lax.fori_loop(0, NBLOCKS, zb, 0)

  zero_acc_slice()

  def zc(r, c):
    cntblk_v[r, :] = zeros16
    return c
  lax.fori_loop(0, NB, zc, 0)

  def zcb(b, c):
    pltpu.sync_copy(cntblk_v, cnt_sh.at[pl.ds(base_n + b * NB, NB)])
    return c
  lax.fori_loop(0, NBLOCKS, zcb, 0)

  def fo(r, c):
    ones_v[r, :] = ones16
    return c
  lax.fori_loop(0, CHUNK, fo, 0)

  pltpu.sync_copy(
      x2.at[pl.ds(cid * N_PAD + base_n, ROWS_PER_TILE)],
      table_sh.at[pl.ds(base_n, ROWS_PER_TILE)])
  plsc.subcore_barrier()

  def edge_loop(count_too):
    def ch_body(ch, c):
      e0 = base_e + ch * CHUNK
      pltpu.sync_copy(src_hbm.at[pl.ds(e0, CHUNK)], src_v)
      pltpu.sync_copy(dst_hbm.at[pl.ds(e0, CHUNK)], dst_v)
      pltpu.async_copy(table_sh.at[src_v], rows_v, sem).wait()
      if count_too:
        pltpu.sync_copy(ones_v, cnt_sh.at[dst_v], add=True)
      pltpu.sync_copy(rows_v, acc_sh.at[dst_v], add=True)
      return c
    lax.fori_loop(0, NUM_CHUNKS, ch_body, 0)

  edge_loop(count_too=True)
  plsc.subcore_barrier()

  def normalize(dst_ref, dst_base):
    def blk(b, c):
      row0 = base_n + b * NB
      pltpu.sync_copy(acc_sh.at[pl.ds(row0, NB)], norm_v)
      pltpu.sync_copy(cnt_sh.at[pl.ds(row0, NB)], cntblk_v)

      def nr(r, cc):
        iv16 = 1.0 / jnp.maximum(cntblk_v[r, :], 1.0)
        for j in range(DH // LANES):
          sl = pl.ds(j * LANES, LANES)
          norm_v[r, sl] = norm_v[r, sl] * iv16
        return cc
      lax.fori_loop(0, NB, nr, 0)
      pltpu.sync_copy(norm_v, dst_ref.at[pl.ds(dst_base + b * NB, NB)])
      return c
    lax.fori_loop(0, NBLOCKS, blk, 0)

  normalize(table_sh, base_n)
  zero_acc_slice()
  plsc.subcore_barrier()

  edge_loop(count_too=False)
  plsc.subcore_barrier()

  normalize(out_hbm, cid * N_PAD + base_n)


@jax.jit
def kernel(x, edge_index):
  x2 = jnp.zeros((NUM_CORES * N_PAD, DH), jnp.float32)
  x2 = x2.at[:N].set(x[:, :DH]).at[N_PAD:N_PAD + N].set(x[:, DH:])

  pad = E_PAD - E
  pad_ids = jnp.arange(pad, dtype=jnp.int32)
  src = jnp.concatenate([edge_index[0], pad_ids % N])
  dst = jnp.concatenate([edge_index[1], N + pad_ids % (N_PAD - N)])

  mesh = plsc.VectorSubcoreMesh(core_axis_name="c", subcore_axis_name="s")
  out = pl.kernel(
      _body,
      out_type=jax.ShapeDtypeStruct((NUM_CORES * N_PAD, DH), jnp.float32),
      mesh=mesh,
      compiler_params=pltpu.CompilerParams(use_tc_tiling_on_sc=False),
      scratch_types=[
          pltpu.VMEM_SHARED((N_PAD, DH), jnp.float32),
          pltpu.VMEM_SHARED((N_PAD, DH), jnp.float32),
          pltpu.VMEM_SHARED((N_PAD, CNT_W), jnp.float32),
          pltpu.VMEM((CHUNK,), jnp.int32),
          pltpu.VMEM((CHUNK,), jnp.int32),
          pltpu.VMEM((CHUNK, DH), jnp.float32),
          pltpu.VMEM((CHUNK, CNT_W), jnp.float32),
          pltpu.VMEM((NB, CNT_W), jnp.float32),
          pltpu.VMEM((NB, DH), jnp.float32),
          pltpu.SemaphoreType.DMA,
      ],
  )(x2, src, dst)

  return jnp.concatenate([out[:N], out[N_PAD:N_PAD + N]], axis=1)


# speedup vs baseline: 7.3931x; 1.3839x over previous
import jax
import jax.numpy as jnp
from jax import lax
from jax.experimental import pallas as pl
from jax.experimental.pallas import tpu as pltpu
from jax.experimental.pallas import tpu_sc as plsc

N = 10000
D = 128
E = 320000

NUM_CORES = 2
NUM_SUBCORES = 16
LANES = 16
DH = D // NUM_CORES
N_PAD = 10240
ROWS_PER_TILE = N_PAD // NUM_SUBCORES
NB = 32
NBLOCKS = ROWS_PER_TILE // NB
CHUNK = 128
E_PAD = 327680
EDGES_PER_TILE = E_PAD // NUM_SUBCORES
NUM_CHUNKS = EDGES_PER_TILE // CHUNK
CNT_W = 16


def _body(x2, src2d, dst2d, zrows_hbm, zcnt_hbm, ones_hbm, out_hbm,
          table_sh, acc_sh, cnt_sh,
          srcp_v, dstp_v, rows0_v, rows1_v, rows2_v, rows3_v,
          ones_v, cntblk_v, norm_v,
          gs0, gs1, gs2, gs3, as0, as1, as2, as3, cs0, cs1, cs2, cs3):
  cid = lax.axis_index("c")
  sid = lax.axis_index("s")
  base_n = sid * ROWS_PER_TILE
  base_ch = sid * NUM_CHUNKS

  def zero_acc_slice():
    pltpu.sync_copy(zrows_hbm, acc_sh.at[pl.ds(base_n, ROWS_PER_TILE)])

  zero_acc_slice()
  pltpu.sync_copy(zcnt_hbm, cnt_sh.at[pl.ds(base_n, ROWS_PER_TILE)])
  pltpu.sync_copy(ones_hbm, ones_v)
  pltpu.sync_copy(
      x2.at[pl.ds(cid * N_PAD + base_n, ROWS_PER_TILE)],
      table_sh.at[pl.ds(base_n, ROWS_PER_TILE)])
  plsc.subcore_barrier()

  def edge_loop(count_too):
    def quad_body(q, c):
      ch0 = base_ch + q * 4
      pltpu.sync_copy(src2d.at[pl.ds(ch0, 4)], srcp_v)
      pltpu.sync_copy(dst2d.at[pl.ds(ch0, 4)], dstp_v)
      g0 = pltpu.async_copy(table_sh.at[srcp_v.at[0]], rows0_v, gs0)
      g1 = pltpu.async_copy(table_sh.at[srcp_v.at[1]], rows1_v, gs1)
      if count_too:
        c0 = pltpu.async_copy(ones_v, cnt_sh.at[dstp_v.at[0]], cs0, add=True)
        c1 = pltpu.async_copy(ones_v, cnt_sh.at[dstp_v.at[1]], cs1, add=True)
        c2 = pltpu.async_copy(ones_v, cnt_sh.at[dstp_v.at[2]], cs2, add=True)
        c3 = pltpu.async_copy(ones_v, cnt_sh.at[dstp_v.at[3]], cs3, add=True)
      g0.wait()
      a0 = pltpu.async_copy(rows0_v, acc_sh.at[dstp_v.at[0]], as0, add=True)
      g1.wait()
      a1 = pltpu.async_copy(rows1_v, acc_sh.at[dstp_v.at[1]], as1, add=True)
      g2 = pltpu.async_copy(table_sh.at[srcp_v.at[2]], rows2_v, gs2)
      g3 = pltpu.async_copy(table_sh.at[srcp_v.at[3]], rows3_v, gs3)
      a0.wait()
      a1.wait()
      g2.wait()
      a2 = pltpu.async_copy(rows2_v, acc_sh.at[dstp_v.at[2]], as2, add=True)
      g3.wait()
      a3 = pltpu.async_copy(rows3_v, acc_sh.at[dstp_v.at[3]], as3, add=True)
      if count_too:
        c0.wait()
        c1.wait()
        c2.wait()
        c3.wait()
      a2.wait()
      a3.wait()
      return c
    lax.fori_loop(0, NUM_CHUNKS // 4, quad_body, 0)

  edge_loop(count_too=True)
  plsc.subcore_barrier()

  def normalize(dst_ref, dst_base):
    def blk(b, c):
      row0 = base_n + b * NB
      pltpu.sync_copy(acc_sh.at[pl.ds(row0, NB)], norm_v)
      pltpu.sync_copy(cnt_sh.at[pl.ds(row0, NB)], cntblk_v)

      def nr(r, cc):
        iv16 = 1.0 / jnp.maximum(cntblk_v[r, :], 1.0)
        for j in range(DH // LANES):
          sl = pl.ds(j * LANES, LANES)
          norm_v[r, sl] = norm_v[r, sl] * iv16
        return cc
      lax.fori_loop(0, NB, nr, 0)
      pltpu.sync_copy(norm_v, dst_ref.at[pl.ds(dst_base + b * NB, NB)])
      return c
    lax.fori_loop(0, NBLOCKS, blk, 0)

  normalize(table_sh, base_n)
  zero_acc_slice()
  plsc.subcore_barrier()

  edge_loop(count_too=False)
  plsc.subcore_barrier()

  normalize(out_hbm, cid * N_PAD + base_n)


@jax.jit
def kernel(x, edge_index):
  x2 = jnp.zeros((NUM_CORES * N_PAD, DH), jnp.float32)
  x2 = x2.at[:N].set(x[:, :DH]).at[N_PAD:N_PAD + N].set(x[:, DH:])

  pad = E_PAD - E
  pad_ids = jnp.arange(pad, dtype=jnp.int32)
  src = jnp.concatenate([edge_index[0], pad_ids % N]).reshape(-1, CHUNK)
  dst = jnp.concatenate(
      [edge_index[1], N + pad_ids % (N_PAD - N)]).reshape(-1, CHUNK)

  zrows = jnp.zeros((ROWS_PER_TILE, DH), jnp.float32)
  zcnt = jnp.zeros((ROWS_PER_TILE, CNT_W), jnp.float32)
  ones = jnp.ones((CHUNK, CNT_W), jnp.float32)

  mesh = plsc.VectorSubcoreMesh(core_axis_name="c", subcore_axis_name="s")
  out = pl.kernel(
      _body,
      out_type=jax.ShapeDtypeStruct((NUM_CORES * N_PAD, DH), jnp.float32),
      mesh=mesh,
      compiler_params=pltpu.CompilerParams(use_tc_tiling_on_sc=False),
      scratch_types=[
          pltpu.VMEM_SHARED((N_PAD, DH), jnp.float32),
          pltpu.VMEM_SHARED((N_PAD, DH), jnp.float32),
          pltpu.VMEM_SHARED((N_PAD, CNT_W), jnp.float32),
          pltpu.VMEM((4, CHUNK), jnp.int32),
          pltpu.VMEM((4, CHUNK), jnp.int32),
          pltpu.VMEM((CHUNK, DH), jnp.float32),
          pltpu.VMEM((CHUNK, DH), jnp.float32),
          pltpu.VMEM((CHUNK, DH), jnp.float32),
          pltpu.VMEM((CHUNK, DH), jnp.float32),
          pltpu.VMEM((CHUNK, CNT_W), jnp.float32),
          pltpu.VMEM((NB, CNT_W), jnp.float32),
          pltpu.VMEM((NB, DH), jnp.float32),
          pltpu.SemaphoreType.DMA,
          pltpu.SemaphoreType.DMA,
          pltpu.SemaphoreType.DMA,
          pltpu.SemaphoreType.DMA,
          pltpu.SemaphoreType.DMA,
          pltpu.SemaphoreType.DMA,
          pltpu.SemaphoreType.DMA,
          pltpu.SemaphoreType.DMA,
          pltpu.SemaphoreType.DMA,
          pltpu.SemaphoreType.DMA,
          pltpu.SemaphoreType.DMA,
          pltpu.SemaphoreType.DMA,
      ],
  )(x2, src, dst, zrows, zcnt, ones)

  return jnp.concatenate([out[:N], out[N_PAD:N_PAD + N]], axis=1)
